# baseline (speedup 1.0000x reference)
import functools

import jax
import jax.numpy as jnp
from jax import lax
from jax.experimental import pallas as pl
from jax.experimental.pallas import tpu as pltpu
from jax.experimental.pallas import tpu_sc as plsc

_NN = 13


def kernel(num_features, cat_features, tables):
    N = num_features.shape[0]
    F, V, D = tables.shape
    d_out = _NN + F * D

    tab_t = jnp.transpose(tables, (0, 2, 1)).reshape(F * D, V)
    cat_t = jnp.transpose(cat_features, (1, 0)).astype(jnp.int32)
    num_t = jnp.transpose(num_features, (1, 0))

    NW = 32
    SW = F * D // NW
    NC = 4096
    NCH = N // NC

    mesh = plsc.VectorSubcoreMesh(core_axis_name="c", subcore_axis_name="s")

    @functools.partial(
        pl.kernel,
        out_type=jax.ShapeDtypeStruct((d_out, N), jnp.float32),
        mesh=mesh,
        scratch_types=[
            pltpu.VMEM((V,), jnp.float32),
            pltpu.VMEM((N,), jnp.int32),
            pltpu.VMEM((2, NC), jnp.float32),
            pltpu.SemaphoreType.DMA,
            pltpu.SemaphoreType.DMA,
        ],
        compiler_params=pltpu.CompilerParams(
            use_tc_tiling_on_sc=False, needs_layout_passes=False
        ),
    )
    def _embed(tab_hbm, cat_hbm, num_hbm, out_hbm,
               slice_v, cat_v, out_v, sem, osem):
        wid = lax.axis_index("s") * 2 + lax.axis_index("c")

        @pl.when(wid < _NN)
        def _():
            pltpu.sync_copy(num_hbm.at[wid, :], slice_v.at[pl.ds(0, N)])
            pltpu.sync_copy(slice_v.at[pl.ds(0, N)], out_hbm.at[wid, :])

        s0 = wid * SW
        pltpu.async_copy(tab_hbm.at[s0, :], slice_v, sem)

        def slice_body(i, f_loaded):
            s = s0 + i
            f = s // D

            @pl.when(f != f_loaded)
            def _():
                pltpu.sync_copy(cat_hbm.at[f, :], cat_v)

            pltpu.make_async_copy(tab_hbm.at[s, :], slice_v, sem).wait()

            for c in range(NCH):
                n0 = c * NC
                b = c % 2
                if c >= 2:
                    pltpu.make_async_copy(
                        out_v.at[b], out_hbm.at[0, pl.ds(0, NC)], osem
                    ).wait()

                ov = out_v.at[b]
                cv = cat_v.at[pl.ds(n0, NC)]

                def vec_body(i16, _):
                    o = i16 * 16
                    ov[pl.ds(o, 16)] = plsc.load_gather(
                        slice_v, [cv[pl.ds(o, 16)]]
                    )
                    return 0

                lax.fori_loop(0, NC // 16, vec_body, 0, unroll=16)
                pltpu.async_copy(
                    out_v.at[b], out_hbm.at[_NN + s, pl.ds(n0, NC)], osem
                )
            @pl.when(i < SW - 1)
            def _():
                pltpu.async_copy(tab_hbm.at[s + 1, :], slice_v, sem)

            for c in range(NCH - 2, NCH):
                b = c % 2
                pltpu.make_async_copy(
                    out_v.at[b], out_hbm.at[0, pl.ds(0, NC)], osem
                ).wait()
            return f

        lax.fori_loop(0, SW, slice_body, jnp.int32(-1), unroll=False)

    out_t = _embed(tab_t, cat_t, num_t)
    return jnp.transpose(out_t, (1, 0))


# speedup vs baseline: 1.0017x; 1.0017x over previous
import functools

import jax
import jax.numpy as jnp
from jax import lax
from jax.experimental import pallas as pl
from jax.experimental.pallas import tpu as pltpu
from jax.experimental.pallas import tpu_sc as plsc

_NN = 13


def kernel(num_features, cat_features, tables):
    N = num_features.shape[0]
    F, V, D = tables.shape
    d_out = _NN + F * D

    tab_t = jnp.transpose(tables, (0, 2, 1)).reshape(F * D, V)
    cat_t = jnp.transpose(cat_features, (1, 0)).astype(jnp.int32)
    num_t = jnp.transpose(num_features, (1, 0))

    NW = 32
    SW = F * D // NW
    NC = 2048
    NCH = N // NC

    mesh = plsc.VectorSubcoreMesh(core_axis_name="c", subcore_axis_name="s")

    @functools.partial(
        pl.kernel,
        out_type=jax.ShapeDtypeStruct((d_out, N), jnp.float32),
        mesh=mesh,
        scratch_types=[
            pltpu.VMEM((V,), jnp.float32),
            pltpu.VMEM((N,), jnp.int32),
            pltpu.VMEM((4, NC), jnp.float32),
            pltpu.SemaphoreType.DMA,
            pltpu.SemaphoreType.DMA,
        ],
        compiler_params=pltpu.CompilerParams(
            use_tc_tiling_on_sc=False, needs_layout_passes=False
        ),
    )
    def _embed(tab_hbm, cat_hbm, num_hbm, out_hbm,
               slice_v, cat_v, out_v, sem, osem):
        wid = lax.axis_index("s") * 2 + lax.axis_index("c")

        @pl.when(wid < _NN)
        def _():
            pltpu.sync_copy(num_hbm.at[wid, :], slice_v.at[pl.ds(0, N)])
            pltpu.sync_copy(slice_v.at[pl.ds(0, N)], out_hbm.at[wid, :])

        s0 = wid * SW
        pltpu.async_copy(tab_hbm.at[s0, :], slice_v, sem)

        def slice_body(i, f_loaded):
            s = s0 + i
            f = s // D

            @pl.when(f != f_loaded)
            def _():
                pltpu.sync_copy(cat_hbm.at[f, :], cat_v)

            pltpu.make_async_copy(tab_hbm.at[s, :], slice_v, sem).wait()

            for c in range(NCH):
                n0 = c * NC
                b = c % 4
                if c >= 4:
                    pltpu.make_async_copy(
                        out_v.at[b], out_hbm.at[0, pl.ds(0, NC)], osem
                    ).wait()

                ov = out_v.at[b]
                cv = cat_v.at[pl.ds(n0, NC)]

                def vec_body(i16, _):
                    o = i16 * 16
                    ov[pl.ds(o, 16)] = plsc.load_gather(
                        slice_v, [cv[pl.ds(o, 16)]]
                    )
                    return 0

                lax.fori_loop(0, NC // 16, vec_body, 0, unroll=16)
                pltpu.async_copy(
                    out_v.at[b], out_hbm.at[_NN + s, pl.ds(n0, NC)], osem
                )
            @pl.when(i < SW - 1)
            def _():
                pltpu.async_copy(tab_hbm.at[s + 1, :], slice_v, sem)

            for c in range(NCH - 4, NCH):
                b = c % 4
                pltpu.make_async_copy(
                    out_v.at[b], out_hbm.at[0, pl.ds(0, NC)], osem
                ).wait()
            return f

        lax.fori_loop(0, SW, slice_body, jnp.int32(-1), unroll=False)

    out_t = _embed(tab_t, cat_t, num_t)
    return jnp.transpose(out_t, (1, 0))
